# Initial kernel scaffold; baseline (speedup 1.0000x reference)
import functools

import jax
import jax.numpy as jnp
from jax import lax
from jax.experimental import pallas as pl
from jax.experimental.pallas import tpu as pltpu
from jax.experimental.pallas import tpu_sc as plsc

_N = 10000
_E = 320000
_D = 128
_C = 16

_NC = 2
_NS = 16
_EPC = _E // _NC
_EPT = _EPC // _NS
_CH = 80
_NCHUNK = _EPT // _CH
_RPT = _N // _NS

_mesh = plsc.VectorSubcoreMesh(core_axis_name="c", subcore_axis_name="s")


def _deg_body(src_hbm, dst_hbm, ones_hbm, zeros_hbm, out_hbm,
              idx_v, ones_v, acc_s, acc_d):
  c = lax.axis_index("c")
  s = lax.axis_index("s")
  r0 = s * _RPT
  pltpu.sync_copy(zeros_hbm.at[pl.ds(r0, _RPT)], acc_s.at[pl.ds(r0, _RPT)])
  pltpu.sync_copy(zeros_hbm.at[pl.ds(r0, _RPT)], acc_d.at[pl.ds(r0, _RPT)])
  pltpu.sync_copy(ones_hbm, ones_v)
  plsc.subcore_barrier()
  base0 = c * _EPC + s * _EPT

  @pl.loop(0, _NCHUNK)
  def _(i):
    b = base0 + i * _CH
    pltpu.sync_copy(src_hbm.at[pl.ds(b, _CH)], idx_v)
    pltpu.sync_copy(ones_v, acc_s.at[idx_v], add=True)
    pltpu.sync_copy(dst_hbm.at[pl.ds(b, _CH)], idx_v)
    pltpu.sync_copy(ones_v, acc_d.at[idx_v], add=True)

  plsc.subcore_barrier()
  pltpu.sync_copy(acc_s.at[pl.ds(r0, _RPT)], out_hbm.at[c, 0, pl.ds(r0, _RPT)])
  pltpu.sync_copy(acc_d.at[pl.ds(r0, _RPT)], out_hbm.at[c, 1, pl.ds(r0, _RPT)])


_deg_kernel = pl.kernel(
    _deg_body,
    out_type=jax.ShapeDtypeStruct((_NC, 2, _N, 16), jnp.float32),
    mesh=_mesh,
    scratch_types=[
        pltpu.VMEM((_CH,), jnp.int32),
        pltpu.VMEM((_CH, 16), jnp.float32),
        pltpu.VMEM_SHARED((_N, 16), jnp.float32),
        pltpu.VMEM_SHARED((_N, 16), jnp.float32),
    ],
)


def _segsum_body(feat_hbm, src_hbm, dst_hbm, zeros_hbm, out_hbm,
                 sidx, didx, rows, acc, sem):
  c = lax.axis_index("c")
  s = lax.axis_index("s")
  r0 = s * _RPT
  pltpu.sync_copy(zeros_hbm.at[pl.ds(r0, _RPT)], acc.at[pl.ds(r0, _RPT)])
  plsc.subcore_barrier()
  base0 = c * _EPC + s * _EPT

  @pl.loop(0, _NCHUNK)
  def _(i):
    b = base0 + i * _CH
    pltpu.sync_copy(src_hbm.at[pl.ds(b, _CH)], sidx)
    pltpu.sync_copy(dst_hbm.at[pl.ds(b, _CH)], didx)
    pltpu.async_copy(feat_hbm.at[sidx], rows, sem).wait()
    pltpu.sync_copy(rows, acc.at[didx], add=True)

  plsc.subcore_barrier()
  pltpu.sync_copy(acc.at[pl.ds(r0, _RPT)], out_hbm.at[c, pl.ds(r0, _RPT)])


def _make_segsum(d):
  return pl.kernel(
      _segsum_body,
      out_type=jax.ShapeDtypeStruct((_NC, _N, d), jnp.float32),
      mesh=_mesh,
      scratch_types=[
          pltpu.VMEM((_CH,), jnp.int32),
          pltpu.VMEM((_CH,), jnp.int32),
          pltpu.VMEM((_CH, d), jnp.float32),
          pltpu.VMEM_SHARED((_N, d), jnp.float32),
          pltpu.SemaphoreType.DMA,
      ],
  )


_segsum_d = _make_segsum(_D)
_segsum_c = _make_segsum(_C)


def _norm_col(p0, p1):
  deg = p0[:, 0:1] + p1[:, 0:1]
  return lax.rsqrt(jnp.maximum(deg, 1.0))


def _prescale_body(in_ref, d0_ref, d1_ref, xs_ref):
  xs_ref[...] = in_ref[...] * _norm_col(d0_ref, d1_ref)


_prescale = pl.pallas_call(
    _prescale_body,
    out_shape=jax.ShapeDtypeStruct((_N, _D), jnp.float32),
)


def _mlp_body(a0, a1, dd0, dd1, ds0, ds1, w1, b1, w2, z_ref):
  nd = _norm_col(dd0, dd1)
  ns = _norm_col(ds0, ds1)
  agg = a0[...] + a1[...]
  h = jnp.dot(agg, w1[...], preferred_element_type=jnp.float32) * nd
  h = jnp.maximum(h + b1[...], 0.0)
  z_ref[...] = jnp.dot(h, w2[...], preferred_element_type=jnp.float32) * ns


_mlp = pl.pallas_call(
    _mlp_body,
    out_shape=jax.ShapeDtypeStruct((_N, _C), jnp.float32),
)


def _readout_body(q0, q1, dd0, dd1, b2, out_ref):
  nd = _norm_col(dd0, dd1)
  h = jnp.maximum((q0[...] + q1[...]) * nd + b2[...], 0.0)
  out_ref[...] = jnp.sum(h, axis=0, keepdims=True)


_readout = pl.pallas_call(
    _readout_body,
    out_shape=jax.ShapeDtypeStruct((1, _C), jnp.float32),
)


@jax.jit
def kernel(in_feat, edge_index, W1, b1, W2, b2):
  src = edge_index[0]
  dst = edge_index[1]
  ones = jnp.ones((_CH, 16), jnp.float32)
  zeros16 = jnp.zeros((_N, 16), jnp.float32)
  zeros128 = jnp.zeros((_N, _D), jnp.float32)

  degpart = _deg_kernel(src, dst, ones, zeros16)
  ds0, ds1 = degpart[0, 0], degpart[1, 0]
  dd0, dd1 = degpart[0, 1], degpart[1, 1]

  xs = _prescale(in_feat, ds0, ds1)
  aggpart = _segsum_d(xs, src, dst, zeros128)
  z = _mlp(aggpart[0], aggpart[1], dd0, dd1, ds0, ds1,
           W1, b1.reshape(1, _D), W2)
  qpart = _segsum_c(z, src, dst, zeros16)
  out = _readout(qpart[0], qpart[1], dd0, dd1, b2.reshape(1, _C))
  return out.reshape(_C)


# speedup vs baseline: 5.1504x; 5.1504x over previous
import functools

import jax
import jax.numpy as jnp
from jax import lax
from jax.experimental import pallas as pl
from jax.experimental.pallas import tpu as pltpu
from jax.experimental.pallas import tpu_sc as plsc

_N = 10000
_E = 320000
_D = 128
_C = 16

_NC = 2
_NS = 16
_EPC = _E // _NC
_EPT = _EPC // _NS
_CH = 80
_NCHUNK = _EPT // _CH
_RPT = 624
_TAIL0 = _NS * _RPT
_TAILN = _N - _TAIL0

_mesh = plsc.VectorSubcoreMesh(core_axis_name="c", subcore_axis_name="s")

_linear_params = pltpu.CompilerParams(use_tc_tiling_on_sc=False)


def _deg_body(src_hbm, dst_hbm, ones_hbm, zeros_hbm, out_hbm,
              idx_v, ones_v, acc_s, acc_d):
  c = lax.axis_index("c")
  s = lax.axis_index("s")
  r0 = s * _RPT
  pltpu.sync_copy(zeros_hbm.at[pl.ds(r0, _RPT)], acc_s.at[pl.ds(r0, _RPT)])
  pltpu.sync_copy(zeros_hbm.at[pl.ds(r0, _RPT)], acc_d.at[pl.ds(r0, _RPT)])

  @pl.when(s == _NS - 1)
  def _():
    pltpu.sync_copy(zeros_hbm.at[pl.ds(_TAIL0, _TAILN)],
                    acc_s.at[pl.ds(_TAIL0, _TAILN)])
    pltpu.sync_copy(zeros_hbm.at[pl.ds(_TAIL0, _TAILN)],
                    acc_d.at[pl.ds(_TAIL0, _TAILN)])

  pltpu.sync_copy(ones_hbm, ones_v)
  plsc.subcore_barrier()
  base0 = c * _EPC + s * _EPT

  @pl.loop(0, _NCHUNK)
  def _(i):
    b = base0 + i * _CH
    pltpu.sync_copy(src_hbm.at[pl.ds(b, _CH)], idx_v)
    pltpu.sync_copy(ones_v, acc_s.at[idx_v], add=True)
    pltpu.sync_copy(dst_hbm.at[pl.ds(b, _CH)], idx_v)
    pltpu.sync_copy(ones_v, acc_d.at[idx_v], add=True)

  plsc.subcore_barrier()
  pltpu.sync_copy(acc_s.at[pl.ds(r0, _RPT)], out_hbm.at[c, 0, pl.ds(r0, _RPT)])
  pltpu.sync_copy(acc_d.at[pl.ds(r0, _RPT)], out_hbm.at[c, 1, pl.ds(r0, _RPT)])

  @pl.when(s == _NS - 1)
  def _():
    pltpu.sync_copy(acc_s.at[pl.ds(_TAIL0, _TAILN)],
                    out_hbm.at[c, 0, pl.ds(_TAIL0, _TAILN)])
    pltpu.sync_copy(acc_d.at[pl.ds(_TAIL0, _TAILN)],
                    out_hbm.at[c, 1, pl.ds(_TAIL0, _TAILN)])


_deg_kernel = pl.kernel(
    _deg_body,
    out_type=jax.ShapeDtypeStruct((_NC, 2, _N, 16), jnp.float32),
    mesh=_mesh,
    scratch_types=[
        pltpu.VMEM((_CH,), jnp.int32),
        pltpu.VMEM((_CH, 16), jnp.float32),
        pltpu.VMEM_SHARED((_N, 16), jnp.float32),
        pltpu.VMEM_SHARED((_N, 16), jnp.float32),
    ],
    compiler_params=_linear_params,
)


def _segsum_body(feat_hbm, src_hbm, dst_hbm, zeros_hbm, out_hbm,
                 sidx, didx, rows, acc, sem):
  c = lax.axis_index("c")
  s = lax.axis_index("s")
  r0 = s * _RPT
  pltpu.sync_copy(zeros_hbm.at[pl.ds(r0, _RPT)], acc.at[pl.ds(r0, _RPT)])

  @pl.when(s == _NS - 1)
  def _():
    pltpu.sync_copy(zeros_hbm.at[pl.ds(_TAIL0, _TAILN)],
                    acc.at[pl.ds(_TAIL0, _TAILN)])

  plsc.subcore_barrier()
  base0 = c * _EPC + s * _EPT

  @pl.loop(0, _NCHUNK)
  def _(i):
    b = base0 + i * _CH
    pltpu.sync_copy(src_hbm.at[pl.ds(b, _CH)], sidx)
    pltpu.sync_copy(dst_hbm.at[pl.ds(b, _CH)], didx)
    pltpu.async_copy(feat_hbm.at[sidx], rows, sem).wait()
    pltpu.sync_copy(rows, acc.at[didx], add=True)

  plsc.subcore_barrier()
  pltpu.sync_copy(acc.at[pl.ds(r0, _RPT)], out_hbm.at[c, pl.ds(r0, _RPT)])

  @pl.when(s == _NS - 1)
  def _():
    pltpu.sync_copy(acc.at[pl.ds(_TAIL0, _TAILN)],
                    out_hbm.at[c, pl.ds(_TAIL0, _TAILN)])


_segsum_d = pl.kernel(
    _segsum_body,
    out_type=jax.ShapeDtypeStruct((_NC, _N, _D), jnp.float32),
    mesh=_mesh,
    scratch_types=[
        pltpu.VMEM((_CH,), jnp.int32),
        pltpu.VMEM((_CH,), jnp.int32),
        pltpu.VMEM((_CH, _D), jnp.float32),
        pltpu.VMEM_SHARED((_N, _D), jnp.float32),
        pltpu.SemaphoreType.DMA,
    ],
)


def _segsum16_body(z_hbm, src_hbm, dst_hbm, zeros_hbm, out_hbm,
                   sidx, didx, rows, zsh, acc, sem):
  c = lax.axis_index("c")
  s = lax.axis_index("s")
  r0 = s * _RPT
  pltpu.sync_copy(z_hbm.at[pl.ds(r0, _RPT)], zsh.at[pl.ds(r0, _RPT)])
  pltpu.sync_copy(zeros_hbm.at[pl.ds(r0, _RPT)], acc.at[pl.ds(r0, _RPT)])

  @pl.when(s == _NS - 1)
  def _():
    pltpu.sync_copy(z_hbm.at[pl.ds(_TAIL0, _TAILN)],
                    zsh.at[pl.ds(_TAIL0, _TAILN)])
    pltpu.sync_copy(zeros_hbm.at[pl.ds(_TAIL0, _TAILN)],
                    acc.at[pl.ds(_TAIL0, _TAILN)])

  plsc.subcore_barrier()
  base0 = c * _EPC + s * _EPT

  @pl.loop(0, _NCHUNK)
  def _(i):
    b = base0 + i * _CH
    pltpu.sync_copy(src_hbm.at[pl.ds(b, _CH)], sidx)
    pltpu.sync_copy(dst_hbm.at[pl.ds(b, _CH)], didx)
    pltpu.async_copy(zsh.at[sidx], rows, sem).wait()
    pltpu.sync_copy(rows, acc.at[didx], add=True)

  plsc.subcore_barrier()
  pltpu.sync_copy(acc.at[pl.ds(r0, _RPT)], out_hbm.at[c, pl.ds(r0, _RPT)])

  @pl.when(s == _NS - 1)
  def _():
    pltpu.sync_copy(acc.at[pl.ds(_TAIL0, _TAILN)],
                    out_hbm.at[c, pl.ds(_TAIL0, _TAILN)])


_segsum_c = pl.kernel(
    _segsum16_body,
    out_type=jax.ShapeDtypeStruct((_NC, _N, _C), jnp.float32),
    mesh=_mesh,
    scratch_types=[
        pltpu.VMEM((_CH,), jnp.int32),
        pltpu.VMEM((_CH,), jnp.int32),
        pltpu.VMEM((_CH, _C), jnp.float32),
        pltpu.VMEM_SHARED((_N, _C), jnp.float32),
        pltpu.VMEM_SHARED((_N, _C), jnp.float32),
        pltpu.SemaphoreType.DMA,
    ],
    compiler_params=_linear_params,
)


def _norm_col(p0, p1):
  deg = p0[:, 0:1] + p1[:, 0:1]
  return lax.rsqrt(jnp.maximum(deg, 1.0))


def _prescale_body(in_ref, d0_ref, d1_ref, xs_ref):
  xs_ref[...] = in_ref[...] * _norm_col(d0_ref, d1_ref)


_prescale = pl.pallas_call(
    _prescale_body,
    out_shape=jax.ShapeDtypeStruct((_N, _D), jnp.float32),
)


def _mlp_body(a0, a1, dd0, dd1, ds0, ds1, w1, b1, w2, z_ref):
  nd = _norm_col(dd0, dd1)
  ns = _norm_col(ds0, ds1)
  agg = a0[...] + a1[...]
  h = jnp.dot(agg, w1[...], preferred_element_type=jnp.float32) * nd
  h = jnp.maximum(h + b1[...], 0.0)
  z_ref[...] = jnp.dot(h, w2[...], preferred_element_type=jnp.float32) * ns


_mlp = pl.pallas_call(
    _mlp_body,
    out_shape=jax.ShapeDtypeStruct((_N, _C), jnp.float32),
)


def _readout_body(q0, q1, dd0, dd1, b2, out_ref):
  nd = _norm_col(dd0, dd1)
  h = jnp.maximum((q0[...] + q1[...]) * nd + b2[...], 0.0)
  out_ref[...] = jnp.sum(h, axis=0, keepdims=True)


_readout = pl.pallas_call(
    _readout_body,
    out_shape=jax.ShapeDtypeStruct((1, _C), jnp.float32),
)


@jax.jit
def kernel(in_feat, edge_index, W1, b1, W2, b2):
  src = edge_index[0]
  dst = edge_index[1]
  ones = jnp.ones((_CH, 16), jnp.float32)
  zeros16 = jnp.zeros((_N, 16), jnp.float32)
  zeros128 = jnp.zeros((_N, _D), jnp.float32)

  degpart = _deg_kernel(src, dst, ones, zeros16)
  ds0, ds1 = degpart[0, 0], degpart[1, 0]
  dd0, dd1 = degpart[0, 1], degpart[1, 1]

  xs = _prescale(in_feat, ds0, ds1)
  aggpart = _segsum_d(xs, src, dst, zeros128)
  z = _mlp(aggpart[0], aggpart[1], dd0, dd1, ds0, ds1,
           W1, b1.reshape(1, _D), W2)
  qpart = _segsum_c(z, src, dst, zeros16)
  out = _readout(qpart[0], qpart[1], dd0, dd1, b2.reshape(1, _C))
  return out.reshape(_C)


# speedup vs baseline: 10.5753x; 2.0533x over previous
import functools

import jax
import jax.numpy as jnp
from jax import lax
from jax.experimental import pallas as pl
from jax.experimental.pallas import tpu as pltpu
from jax.experimental.pallas import tpu_sc as plsc

_N = 10000
_E = 320000
_D = 128
_C = 16

_NC = 2
_NS = 16
_EPC = _E // _NC
_EPT = _EPC // _NS
_CH = 128
_NB = _EPT // _CH
_TE = _EPT - _NB * _CH
_RPT = 624
_TAIL0 = _NS * _RPT
_TAILN = _N - _TAIL0

_mesh = plsc.VectorSubcoreMesh(core_axis_name="c", subcore_axis_name="s")

_linear_params = pltpu.CompilerParams(use_tc_tiling_on_sc=False)


def _istart(src_hbm, dst_hbm, b, sb, db, sem):
  pltpu.make_async_copy(src_hbm.at[pl.ds(b, _CH)], sb, sem).start()
  pltpu.make_async_copy(dst_hbm.at[pl.ds(b, _CH)], db, sem).start()


def _iwait(src_hbm, dst_hbm, sb, db, sem):
  pltpu.make_async_copy(src_hbm.at[pl.ds(0, _CH)], sb, sem).wait()
  pltpu.make_async_copy(dst_hbm.at[pl.ds(0, _CH)], db, sem).wait()


def _deg_body(src_hbm, dst_hbm, ones_hbm, zeros_hbm, out_hbm,
              s0, d0, s1, d1, st, dt, ones_v, ones_t, acc_s, acc_d,
              is0, is1):
  c = lax.axis_index("c")
  s = lax.axis_index("s")
  r0 = s * _RPT
  pltpu.sync_copy(zeros_hbm.at[pl.ds(r0, _RPT)], acc_s.at[pl.ds(r0, _RPT)])
  pltpu.sync_copy(zeros_hbm.at[pl.ds(r0, _RPT)], acc_d.at[pl.ds(r0, _RPT)])

  @pl.when(s == _NS - 1)
  def _():
    pltpu.sync_copy(zeros_hbm.at[pl.ds(_TAIL0, _TAILN)],
                    acc_s.at[pl.ds(_TAIL0, _TAILN)])
    pltpu.sync_copy(zeros_hbm.at[pl.ds(_TAIL0, _TAILN)],
                    acc_d.at[pl.ds(_TAIL0, _TAILN)])

  pltpu.sync_copy(ones_hbm, ones_v)
  pltpu.sync_copy(ones_hbm.at[pl.ds(0, _TE)], ones_t)
  plsc.subcore_barrier()
  base0 = c * _EPC + s * _EPT

  _istart(src_hbm, dst_hbm, base0, s0, d0, is0)
  _istart(src_hbm, dst_hbm, base0 + _CH, s1, d1, is1)

  @pl.loop(0, _NB - 2, step=2)
  def _(i):
    _iwait(src_hbm, dst_hbm, s0, d0, is0)
    pltpu.sync_copy(ones_v, acc_s.at[s0], add=True)
    pltpu.sync_copy(ones_v, acc_d.at[d0], add=True)
    _istart(src_hbm, dst_hbm, base0 + (i + 2) * _CH, s0, d0, is0)
    _iwait(src_hbm, dst_hbm, s1, d1, is1)
    pltpu.sync_copy(ones_v, acc_s.at[s1], add=True)
    pltpu.sync_copy(ones_v, acc_d.at[d1], add=True)
    _istart(src_hbm, dst_hbm, base0 + (i + 3) * _CH, s1, d1, is1)

  _iwait(src_hbm, dst_hbm, s0, d0, is0)
  pltpu.sync_copy(ones_v, acc_s.at[s0], add=True)
  pltpu.sync_copy(ones_v, acc_d.at[d0], add=True)
  _iwait(src_hbm, dst_hbm, s1, d1, is1)
  pltpu.sync_copy(ones_v, acc_s.at[s1], add=True)
  pltpu.sync_copy(ones_v, acc_d.at[d1], add=True)

  bt = base0 + _NB * _CH
  pltpu.sync_copy(src_hbm.at[pl.ds(bt, _TE)], st)
  pltpu.sync_copy(dst_hbm.at[pl.ds(bt, _TE)], dt)
  pltpu.sync_copy(ones_t, acc_s.at[st], add=True)
  pltpu.sync_copy(ones_t, acc_d.at[dt], add=True)

  plsc.subcore_barrier()
  pltpu.sync_copy(acc_s.at[pl.ds(r0, _RPT)], out_hbm.at[c, 0, pl.ds(r0, _RPT)])
  pltpu.sync_copy(acc_d.at[pl.ds(r0, _RPT)], out_hbm.at[c, 1, pl.ds(r0, _RPT)])

  @pl.when(s == _NS - 1)
  def _():
    pltpu.sync_copy(acc_s.at[pl.ds(_TAIL0, _TAILN)],
                    out_hbm.at[c, 0, pl.ds(_TAIL0, _TAILN)])
    pltpu.sync_copy(acc_d.at[pl.ds(_TAIL0, _TAILN)],
                    out_hbm.at[c, 1, pl.ds(_TAIL0, _TAILN)])


_deg_kernel = pl.kernel(
    _deg_body,
    out_type=jax.ShapeDtypeStruct((_NC, 2, _N, 16), jnp.float32),
    mesh=_mesh,
    scratch_types=[
        pltpu.VMEM((_CH,), jnp.int32),
        pltpu.VMEM((_CH,), jnp.int32),
        pltpu.VMEM((_CH,), jnp.int32),
        pltpu.VMEM((_CH,), jnp.int32),
        pltpu.VMEM((_TE,), jnp.int32),
        pltpu.VMEM((_TE,), jnp.int32),
        pltpu.VMEM((_CH, 16), jnp.float32),
        pltpu.VMEM((_TE, 16), jnp.float32),
        pltpu.VMEM_SHARED((_N, 16), jnp.float32),
        pltpu.VMEM_SHARED((_N, 16), jnp.float32),
        pltpu.SemaphoreType.DMA,
        pltpu.SemaphoreType.DMA,
    ],
    compiler_params=_linear_params,
)


def _segsum_pipelined(feat_ref, src_hbm, dst_hbm, base0,
                      s0, d0, s1, d1, st, dt, r0b, r1b, rt, acc,
                      is0, is1, gs0, gs1):
  _istart(src_hbm, dst_hbm, base0, s0, d0, is0)
  _istart(src_hbm, dst_hbm, base0 + _CH, s1, d1, is1)
  _iwait(src_hbm, dst_hbm, s0, d0, is0)
  pltpu.make_async_copy(feat_ref.at[s0], r0b, gs0).start()

  @pl.loop(0, _NB - 2, step=2)
  def _(i):
    _iwait(src_hbm, dst_hbm, s1, d1, is1)
    pltpu.make_async_copy(feat_ref.at[s0], r0b, gs0).wait()
    pltpu.make_async_copy(feat_ref.at[s1], r1b, gs1).start()
    pltpu.sync_copy(r0b, acc.at[d0], add=True)
    _istart(src_hbm, dst_hbm, base0 + (i + 2) * _CH, s0, d0, is0)
    _iwait(src_hbm, dst_hbm, s0, d0, is0)
    pltpu.make_async_copy(feat_ref.at[s1], r1b, gs1).wait()
    pltpu.make_async_copy(feat_ref.at[s0], r0b, gs0).start()
    pltpu.sync_copy(r1b, acc.at[d1], add=True)
    _istart(src_hbm, dst_hbm, base0 + (i + 3) * _CH, s1, d1, is1)

  _iwait(src_hbm, dst_hbm, s1, d1, is1)
  pltpu.make_async_copy(feat_ref.at[s0], r0b, gs0).wait()
  pltpu.make_async_copy(feat_ref.at[s1], r1b, gs1).start()
  pltpu.sync_copy(r0b, acc.at[d0], add=True)
  pltpu.make_async_copy(feat_ref.at[s1], r1b, gs1).wait()
  pltpu.sync_copy(r1b, acc.at[d1], add=True)

  bt = base0 + _NB * _CH
  pltpu.sync_copy(src_hbm.at[pl.ds(bt, _TE)], st)
  pltpu.sync_copy(dst_hbm.at[pl.ds(bt, _TE)], dt)
  pltpu.async_copy(feat_ref.at[st], rt, gs0).wait()
  pltpu.sync_copy(rt, acc.at[dt], add=True)


def _segsum_body(feat_hbm, src_hbm, dst_hbm, zeros_hbm, out_hbm,
                 s0, d0, s1, d1, st, dt, r0b, r1b, rt, acc,
                 is0, is1, gs0, gs1):
  c = lax.axis_index("c")
  s = lax.axis_index("s")
  r0 = s * _RPT
  pltpu.sync_copy(zeros_hbm.at[pl.ds(r0, _RPT)], acc.at[pl.ds(r0, _RPT)])

  @pl.when(s == _NS - 1)
  def _():
    pltpu.sync_copy(zeros_hbm.at[pl.ds(_TAIL0, _TAILN)],
                    acc.at[pl.ds(_TAIL0, _TAILN)])

  plsc.subcore_barrier()
  base0 = c * _EPC + s * _EPT
  _segsum_pipelined(feat_hbm, src_hbm, dst_hbm, base0,
                    s0, d0, s1, d1, st, dt, r0b, r1b, rt, acc,
                    is0, is1, gs0, gs1)

  plsc.subcore_barrier()
  pltpu.sync_copy(acc.at[pl.ds(r0, _RPT)], out_hbm.at[c, pl.ds(r0, _RPT)])

  @pl.when(s == _NS - 1)
  def _():
    pltpu.sync_copy(acc.at[pl.ds(_TAIL0, _TAILN)],
                    out_hbm.at[c, pl.ds(_TAIL0, _TAILN)])


_segsum_d = pl.kernel(
    _segsum_body,
    out_type=jax.ShapeDtypeStruct((_NC, _N, _D), jnp.float32),
    mesh=_mesh,
    scratch_types=[
        pltpu.VMEM((_CH,), jnp.int32),
        pltpu.VMEM((_CH,), jnp.int32),
        pltpu.VMEM((_CH,), jnp.int32),
        pltpu.VMEM((_CH,), jnp.int32),
        pltpu.VMEM((_TE,), jnp.int32),
        pltpu.VMEM((_TE,), jnp.int32),
        pltpu.VMEM((_CH, _D), jnp.float32),
        pltpu.VMEM((_CH, _D), jnp.float32),
        pltpu.VMEM((_TE, _D), jnp.float32),
        pltpu.VMEM_SHARED((_N, _D), jnp.float32),
        pltpu.SemaphoreType.DMA,
        pltpu.SemaphoreType.DMA,
        pltpu.SemaphoreType.DMA,
        pltpu.SemaphoreType.DMA,
    ],
)


def _segsum16_body(z_hbm, src_hbm, dst_hbm, zeros_hbm, out_hbm,
                   s0, d0, s1, d1, st, dt, r0b, r1b, rt, zsh, acc,
                   is0, is1, gs0, gs1):
  c = lax.axis_index("c")
  s = lax.axis_index("s")
  r0 = s * _RPT
  pltpu.sync_copy(z_hbm.at[pl.ds(r0, _RPT)], zsh.at[pl.ds(r0, _RPT)])
  pltpu.sync_copy(zeros_hbm.at[pl.ds(r0, _RPT)], acc.at[pl.ds(r0, _RPT)])

  @pl.when(s == _NS - 1)
  def _():
    pltpu.sync_copy(z_hbm.at[pl.ds(_TAIL0, _TAILN)],
                    zsh.at[pl.ds(_TAIL0, _TAILN)])
    pltpu.sync_copy(zeros_hbm.at[pl.ds(_TAIL0, _TAILN)],
                    acc.at[pl.ds(_TAIL0, _TAILN)])

  plsc.subcore_barrier()
  base0 = c * _EPC + s * _EPT
  _segsum_pipelined(zsh, src_hbm, dst_hbm, base0,
                    s0, d0, s1, d1, st, dt, r0b, r1b, rt, acc,
                    is0, is1, gs0, gs1)

  plsc.subcore_barrier()
  pltpu.sync_copy(acc.at[pl.ds(r0, _RPT)], out_hbm.at[c, pl.ds(r0, _RPT)])

  @pl.when(s == _NS - 1)
  def _():
    pltpu.sync_copy(acc.at[pl.ds(_TAIL0, _TAILN)],
                    out_hbm.at[c, pl.ds(_TAIL0, _TAILN)])


_segsum_c = pl.kernel(
    _segsum16_body,
    out_type=jax.ShapeDtypeStruct((_NC, _N, _C), jnp.float32),
    mesh=_mesh,
    scratch_types=[
        pltpu.VMEM((_CH,), jnp.int32),
        pltpu.VMEM((_CH,), jnp.int32),
        pltpu.VMEM((_CH,), jnp.int32),
        pltpu.VMEM((_CH,), jnp.int32),
        pltpu.VMEM((_TE,), jnp.int32),
        pltpu.VMEM((_TE,), jnp.int32),
        pltpu.VMEM((_CH, _C), jnp.float32),
        pltpu.VMEM((_CH, _C), jnp.float32),
        pltpu.VMEM((_TE, _C), jnp.float32),
        pltpu.VMEM_SHARED((_N, _C), jnp.float32),
        pltpu.VMEM_SHARED((_N, _C), jnp.float32),
        pltpu.SemaphoreType.DMA,
        pltpu.SemaphoreType.DMA,
        pltpu.SemaphoreType.DMA,
        pltpu.SemaphoreType.DMA,
    ],
    compiler_params=_linear_params,
)


def _norm_col(p0, p1):
  deg = p0[:, 0:1] + p1[:, 0:1]
  return lax.rsqrt(jnp.maximum(deg, 1.0))


def _prescale_body(in_ref, d0_ref, d1_ref, xs_ref):
  xs_ref[...] = in_ref[...] * _norm_col(d0_ref, d1_ref)


_prescale = pl.pallas_call(
    _prescale_body,
    out_shape=jax.ShapeDtypeStruct((_N, _D), jnp.float32),
)


def _mlp_body(a0, a1, dd0, dd1, ds0, ds1, w1, b1, w2, z_ref):
  nd = _norm_col(dd0, dd1)
  ns = _norm_col(ds0, ds1)
  agg = a0[...] + a1[...]
  h = jnp.dot(agg, w1[...], preferred_element_type=jnp.float32) * nd
  h = jnp.maximum(h + b1[...], 0.0)
  z_ref[...] = jnp.dot(h, w2[...], preferred_element_type=jnp.float32) * ns


_mlp = pl.pallas_call(
    _mlp_body,
    out_shape=jax.ShapeDtypeStruct((_N, _C), jnp.float32),
)


def _readout_body(q0, q1, dd0, dd1, b2, out_ref):
  nd = _norm_col(dd0, dd1)
  h = jnp.maximum((q0[...] + q1[...]) * nd + b2[...], 0.0)
  out_ref[...] = jnp.sum(h, axis=0, keepdims=True)


_readout = pl.pallas_call(
    _readout_body,
    out_shape=jax.ShapeDtypeStruct((1, _C), jnp.float32),
)


@jax.jit
def kernel(in_feat, edge_index, W1, b1, W2, b2):
  src = edge_index[0]
  dst = edge_index[1]
  ones = jnp.ones((_CH, 16), jnp.float32)
  zeros16 = jnp.zeros((_N, 16), jnp.float32)
  zeros128 = jnp.zeros((_N, _D), jnp.float32)

  degpart = _deg_kernel(src, dst, ones, zeros16)
  ds0, ds1 = degpart[0, 0], degpart[1, 0]
  dd0, dd1 = degpart[0, 1], degpart[1, 1]

  xs = _prescale(in_feat, ds0, ds1)
  aggpart = _segsum_d(xs, src, dst, zeros128)
  z = _mlp(aggpart[0], aggpart[1], dd0, dd1, ds0, ds1,
           W1, b1.reshape(1, _D), W2)
  qpart = _segsum_c(z, src, dst, zeros16)
  out = _readout(qpart[0], qpart[1], dd0, dd1, b2.reshape(1, _C))
  return out.reshape(_C)


# speedup vs baseline: 11.2728x; 1.0660x over previous
import functools

import jax
import jax.numpy as jnp
from jax import lax
from jax.experimental import pallas as pl
from jax.experimental.pallas import tpu as pltpu
from jax.experimental.pallas import tpu_sc as plsc

_N = 10000
_E = 320000
_D = 128
_C = 16

_NC = 2
_NS = 16
_EPC = _E // _NC
_EPT = _EPC // _NS
_CH = 128
_NB = _EPT // _CH
_TE = _EPT - _NB * _CH
_RPT = 624
_TAIL0 = _NS * _RPT
_TAILN = _N - _TAIL0

_mesh = plsc.VectorSubcoreMesh(core_axis_name="c", subcore_axis_name="s")

_linear_params = pltpu.CompilerParams(use_tc_tiling_on_sc=False)


def _istart(src_hbm, dst_hbm, b, sb, db, sem):
  pltpu.make_async_copy(src_hbm.at[pl.ds(b, _CH)], sb, sem).start()
  pltpu.make_async_copy(dst_hbm.at[pl.ds(b, _CH)], db, sem).start()


def _iwait(src_hbm, dst_hbm, sb, db, sem):
  pltpu.make_async_copy(src_hbm.at[pl.ds(0, _CH)], sb, sem).wait()
  pltpu.make_async_copy(dst_hbm.at[pl.ds(0, _CH)], db, sem).wait()


def _deg_body(src_hbm, dst_hbm, ones_hbm, zeros_hbm, out_hbm,
              s0, d0, s1, d1, st, dt, ones_v, ones_t, acc_s, acc_d,
              is0, is1):
  c = lax.axis_index("c")
  s = lax.axis_index("s")
  r0 = s * _RPT
  pltpu.sync_copy(zeros_hbm.at[pl.ds(r0, _RPT)], acc_s.at[pl.ds(r0, _RPT)])
  pltpu.sync_copy(zeros_hbm.at[pl.ds(r0, _RPT)], acc_d.at[pl.ds(r0, _RPT)])

  @pl.when(s == _NS - 1)
  def _():
    pltpu.sync_copy(zeros_hbm.at[pl.ds(_TAIL0, _TAILN)],
                    acc_s.at[pl.ds(_TAIL0, _TAILN)])
    pltpu.sync_copy(zeros_hbm.at[pl.ds(_TAIL0, _TAILN)],
                    acc_d.at[pl.ds(_TAIL0, _TAILN)])

  pltpu.sync_copy(ones_hbm, ones_v)
  pltpu.sync_copy(ones_hbm.at[pl.ds(0, _TE)], ones_t)
  plsc.subcore_barrier()
  base0 = c * _EPC + s * _EPT

  _istart(src_hbm, dst_hbm, base0, s0, d0, is0)
  _istart(src_hbm, dst_hbm, base0 + _CH, s1, d1, is1)

  @pl.loop(0, _NB - 2, step=2)
  def _(i):
    _iwait(src_hbm, dst_hbm, s0, d0, is0)
    pltpu.sync_copy(ones_v, acc_s.at[s0], add=True)
    pltpu.sync_copy(ones_v, acc_d.at[d0], add=True)
    _istart(src_hbm, dst_hbm, base0 + (i + 2) * _CH, s0, d0, is0)
    _iwait(src_hbm, dst_hbm, s1, d1, is1)
    pltpu.sync_copy(ones_v, acc_s.at[s1], add=True)
    pltpu.sync_copy(ones_v, acc_d.at[d1], add=True)
    _istart(src_hbm, dst_hbm, base0 + (i + 3) * _CH, s1, d1, is1)

  _iwait(src_hbm, dst_hbm, s0, d0, is0)
  pltpu.sync_copy(ones_v, acc_s.at[s0], add=True)
  pltpu.sync_copy(ones_v, acc_d.at[d0], add=True)
  _iwait(src_hbm, dst_hbm, s1, d1, is1)
  pltpu.sync_copy(ones_v, acc_s.at[s1], add=True)
  pltpu.sync_copy(ones_v, acc_d.at[d1], add=True)

  bt = base0 + _NB * _CH
  pltpu.sync_copy(src_hbm.at[pl.ds(bt, _TE)], st)
  pltpu.sync_copy(dst_hbm.at[pl.ds(bt, _TE)], dt)
  pltpu.sync_copy(ones_t, acc_s.at[st], add=True)
  pltpu.sync_copy(ones_t, acc_d.at[dt], add=True)

  plsc.subcore_barrier()
  pltpu.sync_copy(acc_s.at[pl.ds(r0, _RPT)], out_hbm.at[c, 0, pl.ds(r0, _RPT)])
  pltpu.sync_copy(acc_d.at[pl.ds(r0, _RPT)], out_hbm.at[c, 1, pl.ds(r0, _RPT)])

  @pl.when(s == _NS - 1)
  def _():
    pltpu.sync_copy(acc_s.at[pl.ds(_TAIL0, _TAILN)],
                    out_hbm.at[c, 0, pl.ds(_TAIL0, _TAILN)])
    pltpu.sync_copy(acc_d.at[pl.ds(_TAIL0, _TAILN)],
                    out_hbm.at[c, 1, pl.ds(_TAIL0, _TAILN)])


_deg_kernel = pl.kernel(
    _deg_body,
    out_type=jax.ShapeDtypeStruct((_NC, 2, _N, 16), jnp.float32),
    mesh=_mesh,
    scratch_types=[
        pltpu.VMEM((_CH,), jnp.int32),
        pltpu.VMEM((_CH,), jnp.int32),
        pltpu.VMEM((_CH,), jnp.int32),
        pltpu.VMEM((_CH,), jnp.int32),
        pltpu.VMEM((_TE,), jnp.int32),
        pltpu.VMEM((_TE,), jnp.int32),
        pltpu.VMEM((_CH, 16), jnp.float32),
        pltpu.VMEM((_TE, 16), jnp.float32),
        pltpu.VMEM_SHARED((_N, 16), jnp.float32),
        pltpu.VMEM_SHARED((_N, 16), jnp.float32),
        pltpu.SemaphoreType.DMA,
        pltpu.SemaphoreType.DMA,
    ],
    compiler_params=_linear_params,
)


def _segsum_pipelined(feat_ref, src_hbm, dst_hbm, base0,
                      s0, d0, s1, d1, st, dt, r0b, r1b, rt, acc,
                      is0, is1, gs0, gs1):
  _istart(src_hbm, dst_hbm, base0, s0, d0, is0)
  _istart(src_hbm, dst_hbm, base0 + _CH, s1, d1, is1)
  _iwait(src_hbm, dst_hbm, s0, d0, is0)
  pltpu.make_async_copy(feat_ref.at[s0], r0b, gs0).start()

  @pl.loop(0, _NB - 2, step=2)
  def _(i):
    _iwait(src_hbm, dst_hbm, s1, d1, is1)
    pltpu.make_async_copy(feat_ref.at[s0], r0b, gs0).wait()
    pltpu.make_async_copy(feat_ref.at[s1], r1b, gs1).start()
    pltpu.sync_copy(r0b, acc.at[d0], add=True)
    _istart(src_hbm, dst_hbm, base0 + (i + 2) * _CH, s0, d0, is0)
    _iwait(src_hbm, dst_hbm, s0, d0, is0)
    pltpu.make_async_copy(feat_ref.at[s1], r1b, gs1).wait()
    pltpu.make_async_copy(feat_ref.at[s0], r0b, gs0).start()
    pltpu.sync_copy(r1b, acc.at[d1], add=True)
    _istart(src_hbm, dst_hbm, base0 + (i + 3) * _CH, s1, d1, is1)

  _iwait(src_hbm, dst_hbm, s1, d1, is1)
  pltpu.make_async_copy(feat_ref.at[s0], r0b, gs0).wait()
  pltpu.make_async_copy(feat_ref.at[s1], r1b, gs1).start()
  pltpu.sync_copy(r0b, acc.at[d0], add=True)
  pltpu.make_async_copy(feat_ref.at[s1], r1b, gs1).wait()
  pltpu.sync_copy(r1b, acc.at[d1], add=True)

  bt = base0 + _NB * _CH
  pltpu.sync_copy(src_hbm.at[pl.ds(bt, _TE)], st)
  pltpu.sync_copy(dst_hbm.at[pl.ds(bt, _TE)], dt)
  pltpu.async_copy(feat_ref.at[st], rt, gs0).wait()
  pltpu.sync_copy(rt, acc.at[dt], add=True)


def _segsum_body(feat_hbm, src_hbm, dst_hbm, zeros_hbm, out_hbm,
                 s0, d0, s1, d1, st, dt, r0b, r1b, rt, acc,
                 is0, is1, gs0, gs1):
  c = lax.axis_index("c")
  s = lax.axis_index("s")
  r0 = s * _RPT
  pltpu.sync_copy(zeros_hbm.at[pl.ds(r0, _RPT)], acc.at[pl.ds(r0, _RPT)])

  @pl.when(s == _NS - 1)
  def _():
    pltpu.sync_copy(zeros_hbm.at[pl.ds(_TAIL0, _TAILN)],
                    acc.at[pl.ds(_TAIL0, _TAILN)])

  plsc.subcore_barrier()
  base0 = c * _EPC + s * _EPT
  _segsum_pipelined(feat_hbm, src_hbm, dst_hbm, base0,
                    s0, d0, s1, d1, st, dt, r0b, r1b, rt, acc,
                    is0, is1, gs0, gs1)

  plsc.subcore_barrier()
  pltpu.sync_copy(acc.at[pl.ds(r0, _RPT)], out_hbm.at[c, pl.ds(r0, _RPT)])

  @pl.when(s == _NS - 1)
  def _():
    pltpu.sync_copy(acc.at[pl.ds(_TAIL0, _TAILN)],
                    out_hbm.at[c, pl.ds(_TAIL0, _TAILN)])


_segsum_d = pl.kernel(
    _segsum_body,
    out_type=jax.ShapeDtypeStruct((_NC, _N, _D), jnp.float32),
    mesh=_mesh,
    scratch_types=[
        pltpu.VMEM((_CH,), jnp.int32),
        pltpu.VMEM((_CH,), jnp.int32),
        pltpu.VMEM((_CH,), jnp.int32),
        pltpu.VMEM((_CH,), jnp.int32),
        pltpu.VMEM((_TE,), jnp.int32),
        pltpu.VMEM((_TE,), jnp.int32),
        pltpu.VMEM((_CH, _D), jnp.float32),
        pltpu.VMEM((_CH, _D), jnp.float32),
        pltpu.VMEM((_TE, _D), jnp.float32),
        pltpu.VMEM_SHARED((_N, _D), jnp.float32),
        pltpu.SemaphoreType.DMA,
        pltpu.SemaphoreType.DMA,
        pltpu.SemaphoreType.DMA,
        pltpu.SemaphoreType.DMA,
    ],
)


def _segsum16_body(z_hbm, src_hbm, dst_hbm, zeros_hbm, out_hbm,
                   s0, d0, s1, d1, st, dt, r0b, r1b, rt, zsh, acc,
                   is0, is1, gs0, gs1):
  c = lax.axis_index("c")
  s = lax.axis_index("s")
  r0 = s * _RPT
  pltpu.sync_copy(z_hbm.at[pl.ds(r0, _RPT)], zsh.at[pl.ds(r0, _RPT)])
  pltpu.sync_copy(zeros_hbm.at[pl.ds(r0, _RPT)], acc.at[pl.ds(r0, _RPT)])

  @pl.when(s == _NS - 1)
  def _():
    pltpu.sync_copy(z_hbm.at[pl.ds(_TAIL0, _TAILN)],
                    zsh.at[pl.ds(_TAIL0, _TAILN)])
    pltpu.sync_copy(zeros_hbm.at[pl.ds(_TAIL0, _TAILN)],
                    acc.at[pl.ds(_TAIL0, _TAILN)])

  plsc.subcore_barrier()
  base0 = c * _EPC + s * _EPT
  _segsum_pipelined(zsh, src_hbm, dst_hbm, base0,
                    s0, d0, s1, d1, st, dt, r0b, r1b, rt, acc,
                    is0, is1, gs0, gs1)

  plsc.subcore_barrier()
  pltpu.sync_copy(acc.at[pl.ds(r0, _RPT)], out_hbm.at[c, pl.ds(r0, _RPT)])

  @pl.when(s == _NS - 1)
  def _():
    pltpu.sync_copy(acc.at[pl.ds(_TAIL0, _TAILN)],
                    out_hbm.at[c, pl.ds(_TAIL0, _TAILN)])


_segsum_c = pl.kernel(
    _segsum16_body,
    out_type=jax.ShapeDtypeStruct((_NC, _N, _C), jnp.float32),
    mesh=_mesh,
    scratch_types=[
        pltpu.VMEM((_CH,), jnp.int32),
        pltpu.VMEM((_CH,), jnp.int32),
        pltpu.VMEM((_CH,), jnp.int32),
        pltpu.VMEM((_CH,), jnp.int32),
        pltpu.VMEM((_TE,), jnp.int32),
        pltpu.VMEM((_TE,), jnp.int32),
        pltpu.VMEM((_CH, _C), jnp.float32),
        pltpu.VMEM((_CH, _C), jnp.float32),
        pltpu.VMEM((_TE, _C), jnp.float32),
        pltpu.VMEM_SHARED((_N, _C), jnp.float32),
        pltpu.VMEM_SHARED((_N, _C), jnp.float32),
        pltpu.SemaphoreType.DMA,
        pltpu.SemaphoreType.DMA,
        pltpu.SemaphoreType.DMA,
        pltpu.SemaphoreType.DMA,
    ],
    compiler_params=_linear_params,
)


def _norm_s(deg_ref):
  deg = deg_ref[0, 0, :, 0:1] + deg_ref[1, 0, :, 0:1]
  return lax.rsqrt(jnp.maximum(deg, 1.0))


def _norm_d(deg_ref):
  deg = deg_ref[0, 1, :, 0:1] + deg_ref[1, 1, :, 0:1]
  return lax.rsqrt(jnp.maximum(deg, 1.0))


def _prescale_body(in_ref, deg_ref, xs_ref):
  xs_ref[...] = in_ref[...] * _norm_s(deg_ref)


_prescale = pl.pallas_call(
    _prescale_body,
    out_shape=jax.ShapeDtypeStruct((_N, _D), jnp.float32),
)


def _mlp_body(agg_ref, deg_ref, w1, b1, w2, z_ref):
  nd = _norm_d(deg_ref)
  ns = _norm_s(deg_ref)
  agg = agg_ref[0] + agg_ref[1]
  h = jnp.dot(agg, w1[...], preferred_element_type=jnp.float32) * nd
  h = jnp.maximum(h + b1[...], 0.0)
  z_ref[...] = jnp.dot(h, w2[...], preferred_element_type=jnp.float32) * ns


_mlp = pl.pallas_call(
    _mlp_body,
    out_shape=jax.ShapeDtypeStruct((_N, _C), jnp.float32),
)


def _readout_body(q_ref, deg_ref, b2, out_ref):
  nd = _norm_d(deg_ref)
  h = jnp.maximum((q_ref[0] + q_ref[1]) * nd + b2[...], 0.0)
  out_ref[...] = jnp.sum(h, axis=0, keepdims=True)


_readout = pl.pallas_call(
    _readout_body,
    out_shape=jax.ShapeDtypeStruct((1, _C), jnp.float32),
)


@jax.jit
def kernel(in_feat, edge_index, W1, b1, W2, b2):
  src = edge_index[0]
  dst = edge_index[1]
  ones = jnp.ones((_CH, 16), jnp.float32)
  zeros16 = jnp.zeros((_N, 16), jnp.float32)
  zeros128 = jnp.zeros((_N, _D), jnp.float32)

  degpart = _deg_kernel(src, dst, ones, zeros16)
  xs = _prescale(in_feat, degpart)
  aggpart = _segsum_d(xs, src, dst, zeros128)
  z = _mlp(aggpart, degpart, W1, b1.reshape(1, _D), W2)
  qpart = _segsum_c(z, src, dst, zeros16)
  out = _readout(qpart, degpart, b2.reshape(1, _C))
  return out.reshape(_C)


# speedup vs baseline: 11.4008x; 1.0114x over previous
import functools

import jax
import jax.numpy as jnp
from jax import lax
from jax.experimental import pallas as pl
from jax.experimental.pallas import tpu as pltpu
from jax.experimental.pallas import tpu_sc as plsc

_N = 10000
_E = 320000
_D = 128
_C = 16

_NC = 2
_NS = 16
_EPC = _E // _NC
_EPT = _EPC // _NS
_CH = 128
_NB = _EPT // _CH
_TE = _EPT - _NB * _CH
_RPT = 624
_TAIL0 = _NS * _RPT
_TAILN = _N - _TAIL0

_mesh = plsc.VectorSubcoreMesh(core_axis_name="c", subcore_axis_name="s")

_linear_params = pltpu.CompilerParams(use_tc_tiling_on_sc=False)


def _istart(src_hbm, dst_hbm, b, sb, db, sem):
  pltpu.make_async_copy(src_hbm.at[pl.ds(b, _CH)], sb, sem).start()
  pltpu.make_async_copy(dst_hbm.at[pl.ds(b, _CH)], db, sem).start()


def _iwait(src_hbm, dst_hbm, sb, db, sem):
  pltpu.make_async_copy(src_hbm.at[pl.ds(0, _CH)], sb, sem).wait()
  pltpu.make_async_copy(dst_hbm.at[pl.ds(0, _CH)], db, sem).wait()


def _deg_body(src_hbm, dst_hbm, zeros_hbm, ones_hbm, out_hbm,
              s0, d0, s1, d1, st, dt, ones_b, acc_s, acc_d, is0, is1):
  c = lax.axis_index("c")
  s = lax.axis_index("s")
  r0 = s * _RPT
  pltpu.sync_copy(zeros_hbm.at[pl.ds(r0, _RPT)], acc_s.at[pl.ds(r0, _RPT)])
  pltpu.sync_copy(zeros_hbm.at[pl.ds(r0, _RPT)], acc_d.at[pl.ds(r0, _RPT)])

  @pl.when(s == _NS - 1)
  def _():
    pltpu.sync_copy(zeros_hbm.at[pl.ds(_TAIL0, _TAILN)],
                    acc_s.at[pl.ds(_TAIL0, _TAILN)])
    pltpu.sync_copy(zeros_hbm.at[pl.ds(_TAIL0, _TAILN)],
                    acc_d.at[pl.ds(_TAIL0, _TAILN)])

  pltpu.sync_copy(ones_hbm, ones_b)
  plsc.subcore_barrier()

  base0 = c * _EPC + s * _EPT
  _istart(src_hbm, dst_hbm, base0, s0, d0, is0)
  _istart(src_hbm, dst_hbm, base0 + _CH, s1, d1, is1)

  @pl.loop(0, _NB - 2, step=2)
  def _(i):
    _iwait(src_hbm, dst_hbm, s0, d0, is0)
    pltpu.sync_copy(ones_b, acc_s.at[s0], add=True)
    pltpu.sync_copy(ones_b, acc_d.at[d0], add=True)
    _istart(src_hbm, dst_hbm, base0 + (i + 2) * _CH, s0, d0, is0)
    _iwait(src_hbm, dst_hbm, s1, d1, is1)
    pltpu.sync_copy(ones_b, acc_s.at[s1], add=True)
    pltpu.sync_copy(ones_b, acc_d.at[d1], add=True)
    _istart(src_hbm, dst_hbm, base0 + (i + 3) * _CH, s1, d1, is1)

  _iwait(src_hbm, dst_hbm, s0, d0, is0)
  pltpu.sync_copy(ones_b, acc_s.at[s0], add=True)
  pltpu.sync_copy(ones_b, acc_d.at[d0], add=True)
  _iwait(src_hbm, dst_hbm, s1, d1, is1)
  pltpu.sync_copy(ones_b, acc_s.at[s1], add=True)
  pltpu.sync_copy(ones_b, acc_d.at[d1], add=True)

  bt = base0 + _NB * _CH
  pltpu.sync_copy(src_hbm.at[pl.ds(bt, _TE)], st)
  pltpu.sync_copy(dst_hbm.at[pl.ds(bt, _TE)], dt)
  pltpu.sync_copy(ones_b.at[pl.ds(0, _TE)], acc_s.at[st], add=True)
  pltpu.sync_copy(ones_b.at[pl.ds(0, _TE)], acc_d.at[dt], add=True)

  plsc.subcore_barrier()
  pltpu.sync_copy(acc_s.at[pl.ds(r0, _RPT)], out_hbm.at[c, 0, pl.ds(r0, _RPT)])
  pltpu.sync_copy(acc_d.at[pl.ds(r0, _RPT)], out_hbm.at[c, 1, pl.ds(r0, _RPT)])

  @pl.when(s == _NS - 1)
  def _():
    pltpu.sync_copy(acc_s.at[pl.ds(_TAIL0, _TAILN)],
                    out_hbm.at[c, 0, pl.ds(_TAIL0, _TAILN)])
    pltpu.sync_copy(acc_d.at[pl.ds(_TAIL0, _TAILN)],
                    out_hbm.at[c, 1, pl.ds(_TAIL0, _TAILN)])


_deg_kernel = pl.kernel(
    _deg_body,
    out_type=jax.ShapeDtypeStruct((_NC, 2, _N, _C), jnp.float32),
    mesh=_mesh,
    scratch_types=[
        pltpu.VMEM((_CH,), jnp.int32),
        pltpu.VMEM((_CH,), jnp.int32),
        pltpu.VMEM((_CH,), jnp.int32),
        pltpu.VMEM((_CH,), jnp.int32),
        pltpu.VMEM((_TE,), jnp.int32),
        pltpu.VMEM((_TE,), jnp.int32),
        pltpu.VMEM((_CH, _C), jnp.float32),
        pltpu.VMEM_SHARED((_N, _C), jnp.float32),
        pltpu.VMEM_SHARED((_N, _C), jnp.float32),
        pltpu.SemaphoreType.DMA,
        pltpu.SemaphoreType.DMA,
    ],
    compiler_params=_linear_params,
)


def _segsum_pipelined(feat_ref, src_hbm, dst_hbm, base0,
                      s0, d0, s1, d1, st, dt, r0b, r1b, rt, acc,
                      is0, is1, gs0, gs1):
  _istart(src_hbm, dst_hbm, base0, s0, d0, is0)
  _istart(src_hbm, dst_hbm, base0 + _CH, s1, d1, is1)
  _iwait(src_hbm, dst_hbm, s0, d0, is0)
  pltpu.make_async_copy(feat_ref.at[s0], r0b, gs0).start()

  @pl.loop(0, _NB - 2, step=2)
  def _(i):
    _iwait(src_hbm, dst_hbm, s1, d1, is1)
    pltpu.make_async_copy(feat_ref.at[s0], r0b, gs0).wait()
    pltpu.make_async_copy(feat_ref.at[s1], r1b, gs1).start()
    pltpu.sync_copy(r0b, acc.at[d0], add=True)
    _istart(src_hbm, dst_hbm, base0 + (i + 2) * _CH, s0, d0, is0)
    _iwait(src_hbm, dst_hbm, s0, d0, is0)
    pltpu.make_async_copy(feat_ref.at[s1], r1b, gs1).wait()
    pltpu.make_async_copy(feat_ref.at[s0], r0b, gs0).start()
    pltpu.sync_copy(r1b, acc.at[d1], add=True)
    _istart(src_hbm, dst_hbm, base0 + (i + 3) * _CH, s1, d1, is1)

  _iwait(src_hbm, dst_hbm, s1, d1, is1)
  pltpu.make_async_copy(feat_ref.at[s0], r0b, gs0).wait()
  pltpu.make_async_copy(feat_ref.at[s1], r1b, gs1).start()
  pltpu.sync_copy(r0b, acc.at[d0], add=True)
  pltpu.make_async_copy(feat_ref.at[s1], r1b, gs1).wait()
  pltpu.sync_copy(r1b, acc.at[d1], add=True)

  bt = base0 + _NB * _CH
  pltpu.sync_copy(src_hbm.at[pl.ds(bt, _TE)], st)
  pltpu.sync_copy(dst_hbm.at[pl.ds(bt, _TE)], dt)
  pltpu.async_copy(feat_ref.at[st], rt, gs0).wait()
  pltpu.sync_copy(rt, acc.at[dt], add=True)


def _segsum_body(feat_hbm, src_hbm, dst_hbm, zeros_hbm, out_hbm,
                 s0, d0, s1, d1, st, dt, r0b, r1b, rt, acc,
                 is0, is1, gs0, gs1):
  c = lax.axis_index("c")
  s = lax.axis_index("s")
  r0 = s * _RPT
  pltpu.sync_copy(zeros_hbm.at[pl.ds(r0, _RPT)], acc.at[pl.ds(r0, _RPT)])

  @pl.when(s == _NS - 1)
  def _():
    pltpu.sync_copy(zeros_hbm.at[pl.ds(_TAIL0, _TAILN)],
                    acc.at[pl.ds(_TAIL0, _TAILN)])

  plsc.subcore_barrier()
  base0 = c * _EPC + s * _EPT
  _segsum_pipelined(feat_hbm, src_hbm, dst_hbm, base0,
                    s0, d0, s1, d1, st, dt, r0b, r1b, rt, acc,
                    is0, is1, gs0, gs1)

  plsc.subcore_barrier()
  pltpu.sync_copy(acc.at[pl.ds(r0, _RPT)], out_hbm.at[c, pl.ds(r0, _RPT)])

  @pl.when(s == _NS - 1)
  def _():
    pltpu.sync_copy(acc.at[pl.ds(_TAIL0, _TAILN)],
                    out_hbm.at[c, pl.ds(_TAIL0, _TAILN)])


_segsum_d = pl.kernel(
    _segsum_body,
    out_type=jax.ShapeDtypeStruct((_NC, _N, _D), jnp.float32),
    mesh=_mesh,
    scratch_types=[
        pltpu.VMEM((_CH,), jnp.int32),
        pltpu.VMEM((_CH,), jnp.int32),
        pltpu.VMEM((_CH,), jnp.int32),
        pltpu.VMEM((_CH,), jnp.int32),
        pltpu.VMEM((_TE,), jnp.int32),
        pltpu.VMEM((_TE,), jnp.int32),
        pltpu.VMEM((_CH, _D), jnp.float32),
        pltpu.VMEM((_CH, _D), jnp.float32),
        pltpu.VMEM((_TE, _D), jnp.float32),
        pltpu.VMEM_SHARED((_N, _D), jnp.float32),
        pltpu.SemaphoreType.DMA,
        pltpu.SemaphoreType.DMA,
        pltpu.SemaphoreType.DMA,
        pltpu.SemaphoreType.DMA,
    ],
)


def _segsum16_body(z_hbm, src_hbm, dst_hbm, zeros_hbm, out_hbm,
                   s0, d0, s1, d1, st, dt, r0b, r1b, rt, zsh, acc,
                   is0, is1, gs0, gs1):
  c = lax.axis_index("c")
  s = lax.axis_index("s")
  r0 = s * _RPT
  pltpu.sync_copy(z_hbm.at[pl.ds(r0, _RPT)], zsh.at[pl.ds(r0, _RPT)])
  pltpu.sync_copy(zeros_hbm.at[pl.ds(r0, _RPT)], acc.at[pl.ds(r0, _RPT)])

  @pl.when(s == _NS - 1)
  def _():
    pltpu.sync_copy(z_hbm.at[pl.ds(_TAIL0, _TAILN)],
                    zsh.at[pl.ds(_TAIL0, _TAILN)])
    pltpu.sync_copy(zeros_hbm.at[pl.ds(_TAIL0, _TAILN)],
                    acc.at[pl.ds(_TAIL0, _TAILN)])

  plsc.subcore_barrier()
  base0 = c * _EPC + s * _EPT
  _segsum_pipelined(zsh, src_hbm, dst_hbm, base0,
                    s0, d0, s1, d1, st, dt, r0b, r1b, rt, acc,
                    is0, is1, gs0, gs1)

  plsc.subcore_barrier()
  pltpu.sync_copy(acc.at[pl.ds(r0, _RPT)], out_hbm.at[c, pl.ds(r0, _RPT)])

  @pl.when(s == _NS - 1)
  def _():
    pltpu.sync_copy(acc.at[pl.ds(_TAIL0, _TAILN)],
                    out_hbm.at[c, pl.ds(_TAIL0, _TAILN)])


_segsum_c = pl.kernel(
    _segsum16_body,
    out_type=jax.ShapeDtypeStruct((_NC, _N, _C), jnp.float32),
    mesh=_mesh,
    scratch_types=[
        pltpu.VMEM((_CH,), jnp.int32),
        pltpu.VMEM((_CH,), jnp.int32),
        pltpu.VMEM((_CH,), jnp.int32),
        pltpu.VMEM((_CH,), jnp.int32),
        pltpu.VMEM((_TE,), jnp.int32),
        pltpu.VMEM((_TE,), jnp.int32),
        pltpu.VMEM((_CH, _C), jnp.float32),
        pltpu.VMEM((_CH, _C), jnp.float32),
        pltpu.VMEM((_TE, _C), jnp.float32),
        pltpu.VMEM_SHARED((_N, _C), jnp.float32),
        pltpu.VMEM_SHARED((_N, _C), jnp.float32),
        pltpu.SemaphoreType.DMA,
        pltpu.SemaphoreType.DMA,
        pltpu.SemaphoreType.DMA,
        pltpu.SemaphoreType.DMA,
    ],
    compiler_params=_linear_params,
)


def _xw1_body(in_ref, w1, xw_ref):
  xw_ref[...] = jnp.dot(in_ref[...], w1[...],
                        preferred_element_type=jnp.float32)


_xw1 = pl.pallas_call(
    _xw1_body,
    out_shape=jax.ShapeDtypeStruct((_N, _D), jnp.float32),
)


def _prescale_body(xw_ref, deg_ref, xs_ref, ns_ref, nd_ref):
  deg = deg_ref[0] + deg_ref[1]
  ns = lax.rsqrt(jnp.maximum(deg[0, :, 0], 1.0)).reshape(_N, 1)
  nd = lax.rsqrt(jnp.maximum(deg[1, :, 0], 1.0)).reshape(_N, 1)
  ns_ref[...] = ns
  nd_ref[...] = nd
  xs_ref[...] = xw_ref[...] * ns


_prescale = pl.pallas_call(
    _prescale_body,
    out_shape=[
        jax.ShapeDtypeStruct((_N, _D), jnp.float32),
        jax.ShapeDtypeStruct((_N, 1), jnp.float32),
        jax.ShapeDtypeStruct((_N, 1), jnp.float32),
    ],
)


def _mlp_body(agg_ref, ns_ref, nd_ref, b1, w2, z_ref):
  agg = agg_ref[0] + agg_ref[1]
  h = jnp.maximum(agg * nd_ref[...] + b1[...], 0.0)
  z_ref[...] = jnp.dot(h, w2[...], preferred_element_type=jnp.float32) * ns_ref[...]


_mlp = pl.pallas_call(
    _mlp_body,
    out_shape=jax.ShapeDtypeStruct((_N, _C), jnp.float32),
)


def _readout_body(q_ref, nd_ref, b2, out_ref):
  h = jnp.maximum((q_ref[0] + q_ref[1]) * nd_ref[...] + b2[...], 0.0)
  out_ref[...] = jnp.sum(h, axis=0, keepdims=True)


_readout = pl.pallas_call(
    _readout_body,
    out_shape=jax.ShapeDtypeStruct((1, _C), jnp.float32),
)


@jax.jit
def kernel(in_feat, edge_index, W1, b1, W2, b2):
  src = edge_index[0]
  dst = edge_index[1]
  zeros16 = jnp.zeros((_N, 16), jnp.float32)
  zeros128 = jnp.zeros((_N, _D), jnp.float32)
  ones_rows = jnp.ones((_CH, _C), jnp.float32)

  xw = _xw1(in_feat, W1)
  degpart = _deg_kernel(src, dst, zeros16, ones_rows)
  xs, ns, nd = _prescale(xw, degpart)
  aggpart = _segsum_d(xs, src, dst, zeros128)
  z = _mlp(aggpart, ns, nd, b1.reshape(1, _D), W2)
  qpart = _segsum_c(z, src, dst, zeros16)
  out = _readout(qpart, nd, b2.reshape(1, _C))
  return out.reshape(_C)
